# baseline (speedup 1.0000x reference)
import functools

import jax
import jax.numpy as jnp
from jax import lax
from jax.experimental import pallas as pl
from jax.experimental.pallas import tpu as pltpu
from jax.experimental.pallas import tpu_sc as plsc

_VOCAB = 100000
_EMBED = 200
_BATCH = 1024
_CTX = 50
_HALF = _BATCH // 2

_NC = 2
_NS = 16
_NW = _NC * _NS
_CHUNK_B = 8
_K = 10

_COL_OFFS = tuple(range(0, _EMBED - 16, 16)) + (_EMBED - 16,)
_TAIL0 = _EMBED - 16


def _make_sc(b_lo, nrows):
    bpw = nrows // _NW
    nchunk = bpw // _CHUNK_B

    def _sc_body(x_hbm, tbl_hbm, s_hbm, t_hbm, idx_v, rows_v, out_v, tail_v,
                 *sems):
        wid = lax.axis_index("s") * _NC + lax.axis_index("c")
        b0 = wid * bpw

        def _idx_vecs(e):
            vs = [idx_v[e, pl.ds(c, 16)] for c in (0, 16, 32)]
            vs.append(idx_v[e, pl.ds(e * 0 + 48, 16)])
            return vs

        def chunk_body(chunk, _):
            row0 = b_lo + b0 + chunk * _CHUNK_B
            pltpu.sync_copy(x_hbm.at[pl.ds(row0, _CHUNK_B), :], idx_v)
            v0 = _idx_vecs(wid * 0)
            for k in range(_K):
                pltpu.async_copy(
                    tbl_hbm.at[pl.ds(v0[k // 16][k % 16], 1), :],
                    rows_v.at[k], sems[k])

            def e_body(e, _):
                ve = _idx_vecs(e)
                vn = _idx_vecs(jnp.minimum(e + 1, _CHUNK_B - 1))
                accs = [jnp.zeros((16,), jnp.float32) for _ in _COL_OFFS]
                for i in range(_CTX):
                    k = i % _K
                    pltpu.make_async_copy(
                        tbl_hbm.at[pl.ds(0, 1), :],
                        rows_v.at[k], sems[k]).wait()
                    for j, off in enumerate(_COL_OFFS):
                        accs[j] = accs[j] + rows_v[k, 0, pl.ds(off, 16)]
                    if i < _CTX - _K:
                        c = i + _K
                        pltpu.async_copy(
                            tbl_hbm.at[pl.ds(ve[c // 16][c % 16], 1), :],
                            rows_v.at[k], sems[k])
                    else:
                        c = i + _K - _CTX

                        @pl.when(e < _CHUNK_B - 1)
                        def _issue():
                            pltpu.async_copy(
                                tbl_hbm.at[pl.ds(vn[c // 16][c % 16], 1), :],
                                rows_v.at[k], sems[k])
                row = chunk * _CHUNK_B + e
                for j, off in enumerate(_COL_OFFS[:-1]):
                    out_v[row, pl.ds(off, 16)] = accs[j]
                tail_v[row, :] = accs[-1]
                return 0

            lax.fori_loop(0, _CHUNK_B, e_body, 0)
            return 0

        lax.fori_loop(0, nchunk, chunk_body, 0)
        pltpu.sync_copy(out_v, s_hbm.at[pl.ds(b0, bpw), :])
        pltpu.sync_copy(tail_v, t_hbm.at[pl.ds(b0, bpw), :])

    return functools.partial(
        pl.kernel,
        out_type=(jax.ShapeDtypeStruct((nrows, _EMBED), jnp.float32),
                  jax.ShapeDtypeStruct((nrows, 16), jnp.float32)),
        mesh=plsc.VectorSubcoreMesh(
            core_axis_name="c", subcore_axis_name="s",
            num_cores=_NC, num_subcores=_NS),
        scratch_types=[
            pltpu.VMEM((_CHUNK_B, _CTX), jnp.int32),
            pltpu.VMEM((_K, 1, _EMBED), jnp.float32),
            pltpu.VMEM((bpw, _EMBED), jnp.float32),
            pltpu.VMEM((bpw, 16), jnp.float32),
        ] + [pltpu.SemaphoreType.DMA] * _K,
    )(_sc_body)


_sc_half = tuple(_make_sc(h * _HALF, _HALF) for h in range(2))


_BN = 6656
_BM = 256
_NSLOT = 4
_NN = 15
_NM = _HALF // _BM


def _make_mm_body(m0):
    def _mm_body(s_ref, w_ref, b_ref, o_ref, acc, sem):
        n = pl.program_id(0)
        m = pl.program_id(1)
        step = n * _NM + m
        slot = lax.rem(step, _NSLOT)

        def _copy(sl, row, col):
            return pltpu.make_async_copy(
                acc.at[sl], o_ref.at[pl.ds(row, _BM), pl.ds(col, _BN)],
                sem.at[sl])

        @pl.when(step >= _NSLOT)
        def _wait_prev():
            _copy(slot, (m0 + m) * _BM, 0).wait()

        acc[slot] = lax.dot_general(
            s_ref[...].astype(jnp.bfloat16), w_ref[...].astype(jnp.bfloat16),
            (((1,), (1,)), ((), ())),
            preferred_element_type=jnp.float32,
        ) + b_ref[...]

        _copy(slot, (m0 + m) * _BM, n * _BN).start()

        @pl.when(step == _NN * _NM - 1)
        def _drain():
            for j in range(_NSLOT):
                _copy(j, m0 * _BM, (_NN - 1) * _BN).wait()

    return _mm_body


def _tail_body(o_in, s_ref, w_ref, b_ref, o_ref):
    del o_in
    o_ref[...] = lax.dot_general(
        s_ref[...].astype(jnp.bfloat16), w_ref[...].astype(jnp.bfloat16),
        (((1,), (1,)), ((), ())),
        preferred_element_type=jnp.float32,
    ) + b_ref[...]


def _proj_half(out_prev, s_half, W, b2, h):
    inner = _make_mm_body(h * _NM)
    body = inner
    in_specs = [
        pl.BlockSpec((_BM, _EMBED), lambda n, m: (m, 0)),
        pl.BlockSpec((_BN, _EMBED), lambda n, m: (n, 0)),
        pl.BlockSpec((1, _BN), lambda n, m: (0, n)),
    ]
    args = (s_half, W, b2)
    kwargs = {}
    if out_prev is not None:
        def body2(o_in, *refs):
            del o_in
            return inner(*refs)
        body = body2
        in_specs = [pl.BlockSpec(memory_space=pl.ANY)] + in_specs
        args = (out_prev,) + args
        kwargs["input_output_aliases"] = {0: 0}
    return pl.pallas_call(
        body,
        grid=(_NN, _NM),
        in_specs=in_specs,
        out_specs=pl.BlockSpec(memory_space=pl.ANY),
        out_shape=jax.ShapeDtypeStruct((_BATCH, _VOCAB), jnp.float32),
        scratch_shapes=[
            pltpu.VMEM((_NSLOT, _BM, _BN), jnp.float32),
            pltpu.SemaphoreType.DMA((_NSLOT,)),
        ],
        compiler_params=pltpu.CompilerParams(
            dimension_semantics=("arbitrary", "arbitrary")),
        **kwargs,
    )(*args)


def _proj_tail(out_prev, s, W, b2):
    return pl.pallas_call(
        _tail_body,
        grid=(1,),
        in_specs=[
            pl.BlockSpec(memory_space=pl.ANY),
            pl.BlockSpec((_BATCH, _EMBED), lambda i: (0, 0)),
            pl.BlockSpec((256, _EMBED), lambda i: (_NN * _BN // 256, 0)),
            pl.BlockSpec((1, 256), lambda i: (0, _NN * _BN // 256)),
        ],
        out_specs=pl.BlockSpec((_BATCH, 256),
                               lambda i: (0, _NN * _BN // 256)),
        out_shape=jax.ShapeDtypeStruct((_BATCH, _VOCAB), jnp.float32),
        input_output_aliases={0: 0},
    )(out_prev, s, W, b2)


def _stitch(s_main, s_tail):
    return jnp.concatenate([s_main[:, :_TAIL0 + 8], s_tail[:, 8:]], axis=1)


def kernel(x, emb_table, W, b):
    xi = x.astype(jnp.int32)
    sm0, st0 = _sc_half[0](xi, emb_table)
    sm1, st1 = _sc_half[1](xi, emb_table)
    b2 = b.reshape(1, _VOCAB)
    out = _proj_half(None, _stitch(sm0, st0), W, b2, 0)
    out = _proj_half(out, _stitch(sm1, st1), W, b2, 1)
    s_full = jnp.concatenate(
        [_stitch(sm0, st0), _stitch(sm1, st1)], axis=0)
    return _proj_tail(out, s_full, W, b2)


# speedup vs baseline: 1.0927x; 1.0927x over previous
import functools

import jax
import jax.numpy as jnp
from jax import lax
from jax.experimental import pallas as pl
from jax.experimental.pallas import tpu as pltpu
from jax.experimental.pallas import tpu_sc as plsc

_VOCAB = 100000
_EMBED = 200
_BATCH = 1024
_CTX = 50

_NC = 2
_NS = 16
_NW = _NC * _NS
_BPW = _BATCH // _NW
_CHUNK_B = 8
_NCHUNK = _BPW // _CHUNK_B
_K = 25

_COL_OFFS = tuple(range(0, _EMBED - 16, 16)) + (_EMBED - 16,)
_TAIL0 = _EMBED - 16


def _sc_body(x_hbm, tbl_hbm, s_hbm, t_hbm, idx_v, rows_v, out_v, tail_v,
             *sems):
    wid = lax.axis_index("s") * _NC + lax.axis_index("c")
    b0 = wid * _BPW

    def _idx_vecs(e):
        vs = [idx_v[e, pl.ds(c, 16)] for c in (0, 16, 32)]
        vs.append(idx_v[e, pl.ds(e * 0 + 48, 16)])
        return vs

    def chunk_body(chunk, _):
        row0 = b0 + chunk * _CHUNK_B
        pltpu.sync_copy(x_hbm.at[pl.ds(row0, _CHUNK_B), :], idx_v)
        v0 = _idx_vecs(wid * 0)
        for k in range(_K):
            pltpu.async_copy(
                tbl_hbm.at[pl.ds(v0[k // 16][k % 16], 1), :],
                rows_v.at[k], sems[k])

        def e_body(e, _):
            ve = _idx_vecs(e)
            vn = _idx_vecs(jnp.minimum(e + 1, _CHUNK_B - 1))
            accs = [jnp.zeros((16,), jnp.float32) for _ in _COL_OFFS]
            for i in range(_CTX):
                k = i % _K
                pltpu.make_async_copy(
                    tbl_hbm.at[pl.ds(0, 1), :],
                    rows_v.at[k], sems[k]).wait()
                for j, off in enumerate(_COL_OFFS):
                    accs[j] = accs[j] + rows_v[k, 0, pl.ds(off, 16)]
                if i < _CTX - _K:
                    c = i + _K
                    pltpu.async_copy(
                        tbl_hbm.at[pl.ds(ve[c // 16][c % 16], 1), :],
                        rows_v.at[k], sems[k])
                else:
                    c = i + _K - _CTX

                    @pl.when(e < _CHUNK_B - 1)
                    def _issue():
                        pltpu.async_copy(
                            tbl_hbm.at[pl.ds(vn[c // 16][c % 16], 1), :],
                            rows_v.at[k], sems[k])
            row = chunk * _CHUNK_B + e
            for j, off in enumerate(_COL_OFFS[:-1]):
                out_v[row, pl.ds(off, 16)] = accs[j]
            tail_v[row, :] = accs[-1]
            return 0

        lax.fori_loop(0, _CHUNK_B, e_body, 0)
        return 0

    lax.fori_loop(0, _NCHUNK, chunk_body, 0)
    pltpu.sync_copy(out_v, s_hbm.at[pl.ds(b0, _BPW), :])
    pltpu.sync_copy(tail_v, t_hbm.at[pl.ds(b0, _BPW), :])


_sc_gather_sum = functools.partial(
    pl.kernel,
    out_type=(jax.ShapeDtypeStruct((_BATCH, _EMBED), jnp.float32),
              jax.ShapeDtypeStruct((_BATCH, 16), jnp.float32)),
    mesh=plsc.VectorSubcoreMesh(
        core_axis_name="c", subcore_axis_name="s",
        num_cores=_NC, num_subcores=_NS),
    scratch_types=[
        pltpu.VMEM((_CHUNK_B, _CTX), jnp.int32),
        pltpu.VMEM((_K, 1, _EMBED), jnp.float32),
        pltpu.VMEM((_BPW, _EMBED), jnp.float32),
        pltpu.VMEM((_BPW, 16), jnp.float32),
    ] + [pltpu.SemaphoreType.DMA] * _K,
)(_sc_body)


_BN = 2048


def _mm_body(s_ref, w_ref, b_ref, o_ref):
    o_ref[...] = lax.dot_general(
        s_ref[...].astype(jnp.bfloat16), w_ref[...].astype(jnp.bfloat16),
        (((1,), (1,)), ((), ())),
        preferred_element_type=jnp.float32,
    ) + b_ref[...]


def _projection(s, W, b2):
    return pl.pallas_call(
        _mm_body,
        grid=(pl.cdiv(_VOCAB, _BN),),
        in_specs=[
            pl.BlockSpec((_BATCH, _EMBED), lambda i: (0, 0)),
            pl.BlockSpec((_BN, _EMBED), lambda i: (i, 0)),
            pl.BlockSpec((1, _BN), lambda i: (0, i)),
        ],
        out_specs=pl.BlockSpec((_BATCH, _BN), lambda i: (0, i)),
        out_shape=jax.ShapeDtypeStruct((_BATCH, _VOCAB), jnp.float32),
        compiler_params=pltpu.CompilerParams(
            dimension_semantics=("arbitrary",)),
    )(s, W, b2)


def kernel(x, emb_table, W, b):
    s_main, s_tail = _sc_gather_sum(x.astype(jnp.int32), emb_table)
    s = jnp.concatenate([s_main[:, :_TAIL0 + 8], s_tail[:, 8:]], axis=1)
    return _projection(s, W, b.reshape(1, _VOCAB))


# speedup vs baseline: 1.1071x; 1.0132x over previous
import functools

import jax
import jax.numpy as jnp
from jax import lax
from jax.experimental import pallas as pl
from jax.experimental.pallas import tpu as pltpu
from jax.experimental.pallas import tpu_sc as plsc

_VOCAB = 100000
_EMBED = 200
_BATCH = 1024
_CTX = 50

_NC = 2
_NS = 16
_NW = _NC * _NS
_BPW = _BATCH // _NW
_CHUNK_B = 8
_NCHUNK = _BPW // _CHUNK_B
_K = 10

_COL_OFFS = tuple(range(0, _EMBED - 16, 16)) + (_EMBED - 16,)
_TAIL0 = _EMBED - 16


def _sc_body(x_hbm, tbl_hbm, s_hbm, t_hbm, idx_v, rows_v, out_v, tail_v,
             *sems):
    wid = lax.axis_index("s") * _NC + lax.axis_index("c")
    b0 = wid * _BPW

    def _idx_vecs(e):
        vs = [idx_v[e, pl.ds(c, 16)] for c in (0, 16, 32)]
        vs.append(idx_v[e, pl.ds(e * 0 + 48, 16)])
        return vs

    def chunk_body(chunk, _):
        row0 = b0 + chunk * _CHUNK_B
        pltpu.sync_copy(x_hbm.at[pl.ds(row0, _CHUNK_B), :], idx_v)
        v0 = _idx_vecs(wid * 0)
        for k in range(_K):
            pltpu.async_copy(
                tbl_hbm.at[pl.ds(v0[k // 16][k % 16], 1), :],
                rows_v.at[k], sems[k])

        def e_body(e, _):
            ve = _idx_vecs(e)
            vn = _idx_vecs(jnp.minimum(e + 1, _CHUNK_B - 1))
            accs = [jnp.zeros((16,), jnp.float32) for _ in _COL_OFFS]
            for i in range(_CTX):
                k = i % _K
                pltpu.make_async_copy(
                    tbl_hbm.at[pl.ds(0, 1), :],
                    rows_v.at[k], sems[k]).wait()
                for j, off in enumerate(_COL_OFFS):
                    accs[j] = accs[j] + rows_v[k, 0, pl.ds(off, 16)]
                if i < _CTX - _K:
                    c = i + _K
                    pltpu.async_copy(
                        tbl_hbm.at[pl.ds(ve[c // 16][c % 16], 1), :],
                        rows_v.at[k], sems[k])
                else:
                    c = i + _K - _CTX

                    @pl.when(e < _CHUNK_B - 1)
                    def _issue():
                        pltpu.async_copy(
                            tbl_hbm.at[pl.ds(vn[c // 16][c % 16], 1), :],
                            rows_v.at[k], sems[k])
            row = chunk * _CHUNK_B + e
            for j, off in enumerate(_COL_OFFS[:-1]):
                out_v[row, pl.ds(off, 16)] = accs[j]
            tail_v[row, :] = accs[-1]
            return 0

        lax.fori_loop(0, _CHUNK_B, e_body, 0)
        return 0

    lax.fori_loop(0, _NCHUNK, chunk_body, 0)
    pltpu.sync_copy(out_v, s_hbm.at[pl.ds(b0, _BPW), :])
    pltpu.sync_copy(tail_v, t_hbm.at[pl.ds(b0, _BPW), :])


_sc_gather_sum = functools.partial(
    pl.kernel,
    out_type=(jax.ShapeDtypeStruct((_BATCH, _EMBED), jnp.float32),
              jax.ShapeDtypeStruct((_BATCH, 16), jnp.float32)),
    mesh=plsc.VectorSubcoreMesh(
        core_axis_name="c", subcore_axis_name="s",
        num_cores=_NC, num_subcores=_NS),
    scratch_types=[
        pltpu.VMEM((_CHUNK_B, _CTX), jnp.int32),
        pltpu.VMEM((_K, 1, _EMBED), jnp.float32),
        pltpu.VMEM((_BPW, _EMBED), jnp.float32),
        pltpu.VMEM((_BPW, 16), jnp.float32),
    ] + [pltpu.SemaphoreType.DMA] * _K,
)(_sc_body)


_BN = 2048


def _mm_body(s_ref, w_ref, b_ref, o_ref):
    o_ref[...] = lax.dot_general(
        s_ref[...].astype(jnp.bfloat16), w_ref[...].astype(jnp.bfloat16),
        (((1,), (1,)), ((), ())),
        preferred_element_type=jnp.float32,
    ) + b_ref[...]


def _projection(s, W, b2):
    return pl.pallas_call(
        _mm_body,
        grid=(pl.cdiv(_VOCAB, _BN),),
        in_specs=[
            pl.BlockSpec((_BATCH, _EMBED), lambda i: (0, 0)),
            pl.BlockSpec((_BN, _EMBED), lambda i: (i, 0)),
            pl.BlockSpec((1, _BN), lambda i: (0, i)),
        ],
        out_specs=pl.BlockSpec((_BATCH, _BN), lambda i: (0, i)),
        out_shape=jax.ShapeDtypeStruct((_BATCH, _VOCAB), jnp.float32),
        compiler_params=pltpu.CompilerParams(
            dimension_semantics=("arbitrary",)),
    )(s, W, b2)


def kernel(x, emb_table, W, b):
    s_main, s_tail = _sc_gather_sum(x.astype(jnp.int32), emb_table)
    s = jnp.concatenate([s_main[:, :_TAIL0 + 8], s_tail[:, 8:]], axis=1)
    return _projection(s, W, b.reshape(1, _VOCAB))
